# baseline (speedup 1.0000x reference)
import functools

import jax
import jax.numpy as jnp
from jax import lax
from jax.experimental import pallas as pl
from jax.experimental.pallas import tpu as pltpu
from jax.experimental.pallas import tpu_sc as plsc

D_ = 64
BATCH_ = 4096
HIST_ = 200
NW_ = 32
BL_ = BATCH_ // NW_
NB_ = 4


def _emb_body(words_hbm, table_hbm, out_hbm, widx_v, hidx_v, g_v, t_v,
              gsem, ssem):
    wid = lax.axis_index("s") * 2 + lax.axis_index("c")

    pltpu.sync_copy(words_hbm.at[pl.ds(wid * BL_, BL_)], widx_v)

    iota = lax.iota(jnp.int32, 16)

    def idx_transpose(h):
        hvec = jnp.full((16,), h, jnp.int32)
        for grp in range(8):
            blvec = iota + (grp * 16)
            vals = plsc.load_gather(widx_v, [blvec, hvec])
            hidx_v[h, pl.ds(grp * 16, 16)] = vals

    pl.loop(0, HIST_, step=1)(idx_transpose)

    def gather(h, buf):
        return pltpu.make_async_copy(
            table_hbm.at[hidx_v.at[h]],
            g_v.at[buf],
            gsem,
        )

    def store(h, tb, cb):
        return pltpu.make_async_copy(
            t_v.at[tb, cb],
            out_hbm.at[h, cb, wid],
            ssem,
        )

    for h in range(NB_ - 1):
        gather(h, h).start()

    def step(g):
        for b in range(NB_):
            h = g + b
            tb = b % 2
            gather(h, b).wait()

            @pl.when(h + NB_ - 1 < HIST_)
            def _():
                gather(h + NB_ - 1, (b + NB_ - 1) % NB_).start()

            @pl.when(h >= 2)
            def _():
                for cb in range(8):
                    store(h, tb, cb).wait()

            def tpose(cb):
                for cc in range(8):
                    cvec = jnp.full((16,), cb * 8 + cc, jnp.int32)
                    for grp in range(8):
                        blvec = iota + (grp * 16)
                        vals = plsc.load_gather(g_v.at[b], [blvec, cvec])
                        t_v[tb, cb, cc, pl.ds(grp * 16, 16)] = vals

            pl.loop(0, 8, step=1)(tpose)

            for cb in range(8):
                store(h, tb, cb).start()

    pl.loop(0, HIST_, step=NB_)(step)

    for tb in range(2):
        for cb in range(8):
            store(0, tb, cb).wait()


@jax.jit
def kernel(words, weight):
    mesh = plsc.VectorSubcoreMesh(core_axis_name="c", subcore_axis_name="s")
    out5 = pl.kernel(
        _emb_body,
        mesh=mesh,
        out_type=jax.ShapeDtypeStruct((HIST_, 8, NW_, 8, BL_), jnp.float32),
        scratch_types=[
            pltpu.VMEM((BL_, HIST_), jnp.int32),
            pltpu.VMEM((HIST_, BL_), jnp.int32),
            pltpu.VMEM((NB_, BL_, D_), jnp.float32),
            pltpu.VMEM((2, 8, 8, BL_), jnp.float32),
            pltpu.SemaphoreType.DMA,
            pltpu.SemaphoreType.DMA,
        ],
        compiler_params=pltpu.CompilerParams(
            use_tc_tiling_on_sc=False, needs_layout_passes=False
        ),
    )(words, weight)
    return out5.transpose(2, 4, 0, 1, 3).reshape(BATCH_, HIST_, D_)


# speedup vs baseline: 1.2397x; 1.2397x over previous
import functools

import jax
import jax.numpy as jnp
from jax import lax
from jax.experimental import pallas as pl
from jax.experimental.pallas import tpu as pltpu
from jax.experimental.pallas import tpu_sc as plsc

D_ = 64
BATCH_ = 4096
HIST_ = 200
NW_ = 32
BL_ = BATCH_ // NW_
NB_ = 4


def _emb_body(words_hbm, table_hbm, out_hbm, widx_v, hidx_v, g_v, t_v,
              gsem, ssem):
    wid = lax.axis_index("s") * 2 + lax.axis_index("c")

    pltpu.sync_copy(words_hbm.at[pl.ds(wid * BL_, BL_)], widx_v)

    iota = lax.iota(jnp.int32, 16)

    def idx_transpose(h):
        hvec = jnp.full((16,), h, jnp.int32)
        for grp in range(8):
            blvec = iota + (grp * 16)
            vals = plsc.load_gather(widx_v, [blvec, hvec])
            hidx_v[h, pl.ds(grp * 16, 16)] = vals

    pl.loop(0, HIST_, step=1)(idx_transpose)

    def gather(h, buf):
        return pltpu.make_async_copy(
            table_hbm.at[hidx_v.at[h]],
            g_v.at[buf],
            gsem,
        )

    def store(h, tb):
        return pltpu.make_async_copy(
            t_v.at[tb],
            out_hbm.at[h, :, wid],
            ssem,
        )

    for h in range(NB_ - 1):
        gather(h, h).start()

    def step(g):
        for b in range(NB_):
            h = g + b
            tb = b % 2
            gather(h, b).wait()

            @pl.when(h + NB_ - 1 < HIST_)
            def _():
                gather(h + NB_ - 1, (b + NB_ - 1) % NB_).start()

            @pl.when(h >= 2)
            def _():
                store(h, tb).wait()

            cbvecs = [(iota + cg * 16) // 8 for cg in range(4)]
            ccvecs = [(iota + cg * 16) % 8 for cg in range(4)]
            tbvec = jnp.full((16,), tb, jnp.int32)

            def tpose(bl0):
                for dbl in range(4):
                    blc = bl0 + dbl
                    blvec = jnp.full((16,), blc, jnp.int32)
                    for cg in range(4):
                        vals = g_v[b, blc, pl.ds(cg * 16, 16)]
                        plsc.store_scatter(
                            t_v, [tbvec, cbvecs[cg], ccvecs[cg], blvec], vals
                        )

            pl.loop(0, BL_, step=4)(tpose)

            store(h, tb).start()

    pl.loop(0, HIST_, step=NB_)(step)

    for tb in range(2):
        store(0, tb).wait()


@jax.jit
def kernel(words, weight):
    mesh = plsc.VectorSubcoreMesh(core_axis_name="c", subcore_axis_name="s")
    out5 = pl.kernel(
        _emb_body,
        mesh=mesh,
        out_type=jax.ShapeDtypeStruct((HIST_, 8, NW_, 8, BL_), jnp.float32),
        scratch_types=[
            pltpu.VMEM((BL_, HIST_), jnp.int32),
            pltpu.VMEM((HIST_, BL_), jnp.int32),
            pltpu.VMEM((NB_, BL_, D_), jnp.float32),
            pltpu.VMEM((2, 8, 8, BL_), jnp.float32),
            pltpu.SemaphoreType.DMA,
            pltpu.SemaphoreType.DMA,
        ],
        compiler_params=pltpu.CompilerParams(
            use_tc_tiling_on_sc=False, needs_layout_passes=False
        ),
    )(words, weight)
    return out5.transpose(2, 4, 0, 1, 3).reshape(BATCH_, HIST_, D_)


# speedup vs baseline: 1.9197x; 1.5485x over previous
import functools

import jax
import jax.numpy as jnp
from jax import lax
from jax.experimental import pallas as pl
from jax.experimental.pallas import tpu as pltpu
from jax.experimental.pallas import tpu_sc as plsc

D_ = 64
BATCH_ = 4096
HIST_ = 200
NW_ = 32
BL_ = BATCH_ // NW_
NB_ = 4


def _emb_body(words_hbm, table_hbm, out_hbm, widx_v, hidx_v, g_v, t_v,
              gsem, ssem):
    wid = lax.axis_index("s") * 2 + lax.axis_index("c")

    pltpu.sync_copy(words_hbm.at[pl.ds(wid * BL_, BL_)], widx_v)

    iota = lax.iota(jnp.int32, 16)

    def idx_transpose(h):
        hvec = jnp.full((16,), h, jnp.int32)
        for grp in range(8):
            blvec = iota + (grp * 16)
            vals = plsc.load_gather(widx_v, [blvec, hvec])
            hidx_v[h, pl.ds(grp * 16, 16)] = vals

    pl.loop(0, HIST_, step=1)(idx_transpose)

    def gather(h, buf):
        return pltpu.make_async_copy(
            table_hbm.at[hidx_v.at[h]],
            g_v.at[buf],
            gsem,
        )

    def store(h, tb):
        return pltpu.make_async_copy(
            t_v.at[tb],
            out_hbm.at[h, :, wid],
            ssem,
        )

    for h in range(NB_ - 1):
        gather(h, h).start()

    def step(g):
        for b in range(NB_):
            h = g + b
            tb = b % 2
            gather(h, b).wait()

            @pl.when(h + NB_ - 1 < HIST_)
            def _():
                gather(h + NB_ - 1, (b + NB_ - 1) % NB_).start()

            @pl.when(h >= 2)
            def _():
                store(h, tb).wait()

            def tpose(cb):
                for cc in range(8):
                    cvec = jnp.full((16,), cb * 8 + cc, jnp.int32)
                    for grp in range(8):
                        blvec = iota + (grp * 16)
                        vals = plsc.load_gather(g_v.at[b], [blvec, cvec])
                        t_v[tb, cb, cc, pl.ds(grp * 16, 16)] = vals

            pl.loop(0, 8, step=1)(tpose)

            store(h, tb).start()

    pl.loop(0, HIST_, step=NB_)(step)

    for tb in range(2):
        store(0, tb).wait()


@jax.jit
def kernel(words, weight):
    mesh = plsc.VectorSubcoreMesh(core_axis_name="c", subcore_axis_name="s")
    weight_padded = jnp.pad(weight, ((0, 0), (0, 1)))
    out5 = pl.kernel(
        _emb_body,
        mesh=mesh,
        out_type=jax.ShapeDtypeStruct((HIST_, 8, NW_, 8, BL_), jnp.float32),
        scratch_types=[
            pltpu.VMEM((BL_, HIST_), jnp.int32),
            pltpu.VMEM((HIST_, BL_), jnp.int32),
            pltpu.VMEM((NB_, BL_, D_ + 1), jnp.float32),
            pltpu.VMEM((2, 8, 8, BL_), jnp.float32),
            pltpu.SemaphoreType.DMA,
            pltpu.SemaphoreType.DMA,
        ],
        compiler_params=pltpu.CompilerParams(
            use_tc_tiling_on_sc=False, needs_layout_passes=False
        ),
    )(words, weight_padded)
    return out5.transpose(2, 4, 0, 1, 3).reshape(BATCH_, HIST_, D_)


# speedup vs baseline: 2.8401x; 1.4794x over previous
import functools

import jax
import jax.numpy as jnp
from jax import lax
from jax.experimental import pallas as pl
from jax.experimental.pallas import tpu as pltpu
from jax.experimental.pallas import tpu_sc as plsc

D_ = 64
BATCH_ = 4096
HIST_ = 200
NW_ = 32
BL_ = BATCH_ // NW_
NB_ = 4


def _emb_body(words_hbm, table_hbm, out_hbm, widx_v, hidx_v, g_v, t_v,
              gsem, ssem):
    wid = lax.axis_index("s") * 2 + lax.axis_index("c")

    pltpu.sync_copy(words_hbm.at[pl.ds(wid * BL_, BL_)], widx_v)

    iota = lax.iota(jnp.int32, 16)

    def idx_transpose(h):
        hvec = jnp.full((16,), h, jnp.int32)
        for grp in range(8):
            blvec = iota + (grp * 16)
            vals = plsc.load_gather(widx_v, [blvec, hvec])
            hidx_v[h, pl.ds(grp * 16, 16)] = vals

    pl.loop(0, HIST_, step=1)(idx_transpose)

    def gather(h, buf):
        return pltpu.make_async_copy(
            table_hbm.at[hidx_v.at[h]],
            g_v.at[buf],
            gsem,
        )

    def store(h, tb):
        return pltpu.make_async_copy(
            t_v.at[tb, :, :, pl.ds(0, BL_)],
            out_hbm.at[h, :, wid],
            ssem,
        )

    for h in range(NB_ - 1):
        gather(h, h).start()

    def step(g):
        for b in range(NB_):
            h = g + b
            tb = b % 2
            gather(h, b).wait()

            @pl.when(h + NB_ - 1 < HIST_)
            def _():
                gather(h + NB_ - 1, (b + NB_ - 1) % NB_).start()

            @pl.when(h >= 2)
            def _():
                store(h, tb).wait()

            cbvecs = [(iota + cg * 16) // 8 for cg in range(4)]
            ccvecs = [(iota + cg * 16) % 8 for cg in range(4)]
            tbvec = jnp.full((16,), tb, jnp.int32)

            def tpose(bl0):
                for dbl in range(4):
                    blc = bl0 + dbl
                    blvec = jnp.full((16,), blc, jnp.int32)
                    for cg in range(4):
                        vals = g_v[b, blc, pl.ds(cg * 16, 16)]
                        plsc.store_scatter(
                            t_v, [tbvec, cbvecs[cg], ccvecs[cg], blvec], vals
                        )

            pl.loop(0, BL_, step=4)(tpose)

            store(h, tb).start()

    pl.loop(0, HIST_, step=NB_)(step)

    for tb in range(2):
        store(0, tb).wait()


@jax.jit
def kernel(words, weight):
    mesh = plsc.VectorSubcoreMesh(core_axis_name="c", subcore_axis_name="s")
    out5 = pl.kernel(
        _emb_body,
        mesh=mesh,
        out_type=jax.ShapeDtypeStruct((HIST_, 8, NW_, 8, BL_), jnp.float32),
        scratch_types=[
            pltpu.VMEM((BL_, HIST_), jnp.int32),
            pltpu.VMEM((HIST_, BL_), jnp.int32),
            pltpu.VMEM((NB_, BL_, D_), jnp.float32),
            pltpu.VMEM((2, 8, 8, BL_ + 1), jnp.float32),
            pltpu.SemaphoreType.DMA,
            pltpu.SemaphoreType.DMA,
        ],
        compiler_params=pltpu.CompilerParams(
            use_tc_tiling_on_sc=False, needs_layout_passes=False
        ),
    )(words, weight)
    return out5.transpose(2, 4, 0, 1, 3).reshape(BATCH_, HIST_, D_)


# speedup vs baseline: 5.7631x; 2.0292x over previous
import functools

import jax
import jax.numpy as jnp
from jax import lax
from jax.experimental import pallas as pl
from jax.experimental.pallas import tpu as pltpu
from jax.experimental.pallas import tpu_sc as plsc

D_ = 64
BATCH_ = 4096
HIST_ = 200
NW_ = 32
BL_ = BATCH_ // NW_
NB_ = 4


def _emb_body(words_hbm, table_hbm, out_hbm, widx_v, hidx_v, g_v, t_v,
              gsem, ssem):
    wid = lax.axis_index("s") * 2 + lax.axis_index("c")

    pltpu.sync_copy(words_hbm.at[pl.ds(wid * BL_, BL_)], widx_v)

    iota = lax.iota(jnp.int32, 16)

    def idx_transpose(h):
        hvec = jnp.full((16,), h, jnp.int32)
        for grp in range(8):
            blvec = iota + (grp * 16)
            vals = plsc.load_gather(widx_v, [blvec, hvec])
            hidx_v[h, pl.ds(grp * 16, 16)] = vals

    pl.loop(0, HIST_, step=1)(idx_transpose)

    def gather(h, buf):
        return pltpu.make_async_copy(
            table_hbm.at[hidx_v.at[h]],
            g_v.at[buf],
            gsem,
        )

    def store(h, tb):
        return pltpu.make_async_copy(
            t_v.at[tb, :, :, pl.ds(0, BL_)],
            out_hbm.at[h, :, wid],
            ssem,
        )

    for h in range(NB_ - 1):
        gather(h, h).start()

    def step(g):
        for b in range(NB_):
            h = g + b
            tb = b % 2
            gather(h, b).wait()

            @pl.when(h + NB_ - 1 < HIST_)
            def _():
                gather(h + NB_ - 1, (b + NB_ - 1) % NB_).start()

            @pl.when(h >= 2)
            def _():
                store(h, tb).wait()

            cbvecs = [(iota + cg * 16) // 8 for cg in range(4)]
            ccvecs = [(iota + cg * 16) % 8 for cg in range(4)]
            tbvec = jnp.full((16,), tb, jnp.int32)

            @plsc.parallel_loop(0, BL_, step=4, unroll=2)
            def tpose(bl0):
                for dbl in range(4):
                    blc = bl0 + dbl
                    blvec = jnp.full((16,), blc, jnp.int32)
                    for cg in range(4):
                        vals = g_v[b, blc, pl.ds(cg * 16, 16)]
                        plsc.store_scatter(
                            t_v, [tbvec, cbvecs[cg], ccvecs[cg], blvec], vals
                        )

            store(h, tb).start()

    pl.loop(0, HIST_, step=NB_)(step)

    for tb in range(2):
        store(0, tb).wait()


@jax.jit
def kernel(words, weight):
    mesh = plsc.VectorSubcoreMesh(core_axis_name="c", subcore_axis_name="s")
    out5 = pl.kernel(
        _emb_body,
        mesh=mesh,
        out_type=jax.ShapeDtypeStruct((HIST_, 8, NW_, 8, BL_), jnp.float32),
        scratch_types=[
            pltpu.VMEM((BL_, HIST_), jnp.int32),
            pltpu.VMEM((HIST_, BL_), jnp.int32),
            pltpu.VMEM((NB_, BL_, D_), jnp.float32),
            pltpu.VMEM((2, 8, 8, BL_ + 1), jnp.float32),
            pltpu.SemaphoreType.DMA,
            pltpu.SemaphoreType.DMA,
        ],
        compiler_params=pltpu.CompilerParams(
            use_tc_tiling_on_sc=False, needs_layout_passes=False
        ),
    )(words, weight)
    return out5.transpose(2, 4, 0, 1, 3).reshape(BATCH_, HIST_, D_)
